# Initial kernel scaffold; baseline (speedup 1.0000x reference)
import jax
import jax.numpy as jnp
from jax.experimental import pallas as pl


def kernel(Z, edge_index, edge_weight, W1, b1, W2, b2):
    raise NotImplementedError("write your pallas kernel here")


# speedup vs baseline: 11.8868x; 11.8868x over previous
import functools

import jax
import jax.numpy as jnp
from jax import lax
from jax.experimental import pallas as pl
from jax.experimental.pallas import tpu as pltpu
from jax.experimental.pallas import tpu_sc as plsc

_N = 10000
_E = 320000
_NC = 2
_NS = 16
_NW = _NC * _NS
_EW = _E // _NW
_CHUNK = 80
_NCH = _EW // _CHUNK
_NPAD = 10240
_RPT = _NPAD // _NS
_RB = 1000


def _zero_rows(rows_v, nrows, ncolv):
    z16 = jnp.zeros((16,), jnp.float32)

    def body(r, c):
        for k in range(ncolv):
            rows_v[r, pl.ds(k * 16, 16)] = z16
        return c

    lax.fori_loop(0, nrows, body, 0)


def _zero_acc_slice(rows_v, acc_sh, sid):
    rbase = sid * _RPT
    for q in range(_RPT // _CHUNK):
        pltpu.sync_copy(rows_v, acc_sh.at[pl.ds(rbase + q * _CHUNK, _CHUNK)])


def _make_deg():
    mesh = plsc.VectorSubcoreMesh(core_axis_name="c", subcore_axis_name="s")

    @functools.partial(
        pl.kernel,
        mesh=mesh,
        out_type=jax.ShapeDtypeStruct((_NC, _NPAD, 16), jnp.float32),
        scratch_types=[
            pltpu.VMEM((_CHUNK,), jnp.int32),
            pltpu.VMEM((_CHUNK,), jnp.float32),
            pltpu.VMEM((_CHUNK, 16), jnp.float32),
            pltpu.VMEM_SHARED((_NPAD, 16), jnp.float32),
        ],
    )
    def deg_kernel(dst_hbm, w_hbm, out_hbm, didx_v, w_v, rows_v, acc_sh):
        cid = lax.axis_index("c")
        sid = lax.axis_index("s")
        wid = sid * _NC + cid
        _zero_rows(rows_v, _CHUNK, 1)
        _zero_acc_slice(rows_v, acc_sh, sid)
        plsc.subcore_barrier()

        ebase = wid * _EW

        def chunk(j, c):
            off = ebase + j * _CHUNK
            pltpu.sync_copy(dst_hbm.at[pl.ds(off, _CHUNK)], didx_v)
            pltpu.sync_copy(w_hbm.at[pl.ds(off, _CHUNK)], w_v)
            def fill(g, c2):
                w16 = w_v[pl.ds(g * 16, 16)]
                for l in range(16):
                    rows_v[g * 16 + l, pl.ds(0, 16)] = jnp.full(
                        (16,), w16[l], dtype=jnp.float32)
                return c2

            lax.fori_loop(0, _CHUNK // 16, fill, 0)
            pltpu.sync_copy(rows_v, acc_sh.at[didx_v], add=True)
            return c

        lax.fori_loop(0, _NCH, chunk, 0)
        plsc.subcore_barrier()
        rbase = sid * _RPT
        pltpu.sync_copy(acc_sh.at[pl.ds(rbase, _RPT)],
                        out_hbm.at[cid, pl.ds(rbase, _RPT)])

    return deg_kernel


def _make_prop(D):
    K = D // 16
    mesh = plsc.VectorSubcoreMesh(core_axis_name="c", subcore_axis_name="s")

    @functools.partial(
        pl.kernel,
        mesh=mesh,
        compiler_params=(None if D % 128 == 0 else
                         pltpu.CompilerParams(use_tc_tiling_on_sc=False)),
        out_type=jax.ShapeDtypeStruct((_NC, _NPAD, D), jnp.float32),
        scratch_types=[
            pltpu.VMEM((_CHUNK,), jnp.int32),
            pltpu.VMEM((_CHUNK,), jnp.int32),
            pltpu.VMEM((_CHUNK,), jnp.float32),
            pltpu.VMEM((_CHUNK, D), jnp.float32),
            pltpu.VMEM_SHARED((_NPAD, D), jnp.float32),
            pltpu.SemaphoreType.DMA,
        ],
    )
    def prop(src_hbm, dst_hbm, w_hbm, y_hbm, out_hbm,
             sidx_v, didx_v, w_v, rows_v, acc_sh, sem):
        cid = lax.axis_index("c")
        sid = lax.axis_index("s")
        wid = sid * _NC + cid
        _zero_rows(rows_v, _CHUNK, K)
        _zero_acc_slice(rows_v, acc_sh, sid)
        plsc.subcore_barrier()

        ebase = wid * _EW

        def chunk(j, c):
            off = ebase + j * _CHUNK
            pltpu.sync_copy(src_hbm.at[pl.ds(off, _CHUNK)], sidx_v)
            pltpu.sync_copy(dst_hbm.at[pl.ds(off, _CHUNK)], didx_v)
            pltpu.sync_copy(w_hbm.at[pl.ds(off, _CHUNK)], w_v)
            pltpu.async_copy(y_hbm.at[sidx_v], rows_v, sem).wait()

            def scale(g, c2):
                w16 = w_v[pl.ds(g * 16, 16)]
                for l in range(16):
                    ws = jnp.full((16,), w16[l], dtype=jnp.float32)
                    i = g * 16 + l
                    for k in range(K):
                        rows_v[i, pl.ds(k * 16, 16)] = (
                            rows_v[i, pl.ds(k * 16, 16)] * ws)
                return c2

            lax.fori_loop(0, _CHUNK // 16, scale, 0)
            pltpu.sync_copy(rows_v, acc_sh.at[didx_v], add=True)
            return c

        lax.fori_loop(0, _NCH, chunk, 0)
        plsc.subcore_barrier()
        rbase = sid * _RPT
        pltpu.sync_copy(acc_sh.at[pl.ds(rbase, _RPT)],
                        out_hbm.at[cid, pl.ds(rbase, _RPT)])

    return prop


_deg_call = _make_deg()
_prop128 = _make_prop(128)
_prop32 = _make_prop(32)


def _tc1(degacc, Z):
    def body(dega_ref, z_ref, dinv_ref, y1_ref):
        A = dega_ref[...]
        deg = (jnp.sum(A[0], axis=1) + jnp.sum(A[1], axis=1)) * (1.0 / 16.0) + 1.0
        dinv = lax.rsqrt(deg)[:, None]
        dinv_ref[...] = dinv
        y1_ref[...] = dinv * z_ref[...]

    return pl.pallas_call(
        body,
        grid=(_N // _RB,),
        in_specs=[pl.BlockSpec((2, _RB, 16), lambda i: (0, i, 0)),
                  pl.BlockSpec((_RB, 128), lambda i: (i, 0))],
        out_specs=[pl.BlockSpec((_RB, 1), lambda i: (i, 0)),
                   pl.BlockSpec((_RB, 128), lambda i: (i, 0))],
        out_shape=[jax.ShapeDtypeStruct((_N, 1), jnp.float32),
                   jax.ShapeDtypeStruct((_N, 128), jnp.float32)],
    )(degacc, Z)


def _tc2(p1, y1, dinv, W1, b1r, W2):
    def body(p_ref, y_ref, d_ref, w1_ref, b1_ref, w2_ref, y2_ref):
        dv = d_ref[...]
        pre = dv * (p_ref[0] + p_ref[1] + y_ref[...])
        H = jnp.maximum(
            jnp.dot(pre, w1_ref[...], preferred_element_type=jnp.float32)
            + b1_ref[...], 0.0)
        G = jnp.dot(H, w2_ref[...], preferred_element_type=jnp.float32)
        y2_ref[...] = dv * G

    return pl.pallas_call(
        body,
        grid=(_N // _RB,),
        in_specs=[pl.BlockSpec((2, _RB, 128), lambda i: (0, i, 0)),
                  pl.BlockSpec((_RB, 128), lambda i: (i, 0)),
                  pl.BlockSpec((_RB, 1), lambda i: (i, 0)),
                  pl.BlockSpec((128, 256), lambda i: (0, 0)),
                  pl.BlockSpec((1, 256), lambda i: (0, 0)),
                  pl.BlockSpec((256, 32), lambda i: (0, 0))],
        out_specs=pl.BlockSpec((_RB, 32), lambda i: (i, 0)),
        out_shape=jax.ShapeDtypeStruct((_N, 32), jnp.float32),
    )(p1, y1, dinv, W1, b1r, W2)


def _tc3(p2, y2, dinv, b2r):
    def body(p_ref, y_ref, d_ref, b2_ref, o_ref):
        B = d_ref[...] * (p_ref[0] + p_ref[1] + y_ref[...]) + b2_ref[...]
        m = jnp.max(B, axis=1, keepdims=True)
        e = jnp.exp(B - m)
        o_ref[...] = e / jnp.sum(e, axis=1, keepdims=True)

    return pl.pallas_call(
        body,
        grid=(_N // _RB,),
        in_specs=[pl.BlockSpec((2, _RB, 32), lambda i: (0, i, 0)),
                  pl.BlockSpec((_RB, 32), lambda i: (i, 0)),
                  pl.BlockSpec((_RB, 1), lambda i: (i, 0)),
                  pl.BlockSpec((1, 32), lambda i: (0, 0))],
        out_specs=pl.BlockSpec((_RB, 32), lambda i: (i, 0)),
        out_shape=jax.ShapeDtypeStruct((_N, 32), jnp.float32),
    )(p2, y2, dinv, b2r)


def kernel(Z, edge_index, edge_weight, W1, b1, W2, b2):
    src = edge_index[0]
    dst = edge_index[1]
    degacc = _deg_call(dst, edge_weight)
    dinv, y1 = _tc1(degacc, Z)
    p1 = _prop128(src, dst, edge_weight, y1)
    y2 = _tc2(p1, y1, dinv, W1, b1.reshape(1, -1), W2)
    p2 = _prop32(src, dst, edge_weight, y2)
    return _tc3(p2, y2, dinv, b2.reshape(1, -1))


# speedup vs baseline: 30.0771x; 2.5303x over previous
import functools

import jax
import jax.numpy as jnp
from jax import lax
from jax.experimental import pallas as pl
from jax.experimental.pallas import tpu as pltpu
from jax.experimental.pallas import tpu_sc as plsc

_N = 10000
_E = 320000
_NC = 2
_NS = 16
_NW = _NC * _NS
_EW = _E // _NW
_CHUNK = 80
_NCH = _EW // _CHUNK
_SB = 25
_NSB = _NCH // _SB
_NPAD = 10240
_RPT = _NPAD // _NS
_RB = 1000


def _zero_rows(rows_v, nrows, ncolv):
    z16 = jnp.zeros((16,), jnp.float32)

    def body(r, c):
        for k in range(ncolv):
            rows_v[r, pl.ds(k * 16, 16)] = z16
        return c

    lax.fori_loop(0, nrows, body, 0)


def _zero_acc_slice(rows_v, acc_sh, sid):
    rbase = sid * _RPT
    for q in range(_RPT // _CHUNK):
        pltpu.sync_copy(rows_v, acc_sh.at[pl.ds(rbase + q * _CHUNK, _CHUNK)])


def _make_deg():
    mesh = plsc.VectorSubcoreMesh(core_axis_name="c", subcore_axis_name="s")

    @functools.partial(
        pl.kernel,
        mesh=mesh,
        out_type=jax.ShapeDtypeStruct((_NC, _NPAD, 16), jnp.float32),
        scratch_types=[
            pltpu.VMEM((_SB, _CHUNK), jnp.int32),
            pltpu.VMEM((_SB, _CHUNK), jnp.float32),
            pltpu.VMEM((_CHUNK, 16), jnp.float32),
            pltpu.VMEM_SHARED((_NPAD, 16), jnp.float32),
        ],
    )
    def deg_kernel(dst_hbm, w_hbm, out_hbm, didx_v, w_v, rows_v, acc_sh):
        cid = lax.axis_index("c")
        sid = lax.axis_index("s")
        wid = sid * _NC + cid
        _zero_rows(rows_v, _CHUNK, 1)
        _zero_acc_slice(rows_v, acc_sh, sid)
        plsc.subcore_barrier()

        def block(b, cb):
            pltpu.sync_copy(dst_hbm.at[wid, b], didx_v)
            pltpu.sync_copy(w_hbm.at[wid, b], w_v)

            def chunk(j, c):
                def fill(g, c2):
                    w16 = w_v[j, pl.ds(g * 16, 16)]
                    for l in range(16):
                        rows_v[g * 16 + l, pl.ds(0, 16)] = jnp.full(
                            (16,), w16[l], dtype=jnp.float32)
                    return c2

                lax.fori_loop(0, _CHUNK // 16, fill, 0)
                pltpu.sync_copy(rows_v, acc_sh.at[didx_v.at[j]], add=True)
                return c

            lax.fori_loop(0, _SB, chunk, 0)
            return cb

        lax.fori_loop(0, _NSB, block, 0)
        plsc.subcore_barrier()
        rbase = sid * _RPT
        pltpu.sync_copy(acc_sh.at[pl.ds(rbase, _RPT)],
                        out_hbm.at[cid, pl.ds(rbase, _RPT)])

    return deg_kernel


def _make_prop(D):
    K = D // 16
    mesh = plsc.VectorSubcoreMesh(core_axis_name="c", subcore_axis_name="s")

    @functools.partial(
        pl.kernel,
        mesh=mesh,
        compiler_params=(None if D % 128 == 0 else
                         pltpu.CompilerParams(use_tc_tiling_on_sc=False)),
        out_type=jax.ShapeDtypeStruct((_NC, _NPAD, D), jnp.float32),
        scratch_types=[
            pltpu.VMEM((_SB, _CHUNK), jnp.int32),
            pltpu.VMEM((_SB, _CHUNK), jnp.int32),
            pltpu.VMEM((_SB, _CHUNK), jnp.float32),
            pltpu.VMEM((_CHUNK, D), jnp.float32),
            pltpu.VMEM((_CHUNK, D), jnp.float32),
            pltpu.VMEM_SHARED((_NPAD, D), jnp.float32),
            pltpu.SemaphoreType.DMA,
            pltpu.SemaphoreType.DMA,
        ],
    )
    def prop(src_hbm, dst_hbm, w_hbm, y_hbm, out_hbm,
             sidx_v, didx_v, w_v, rows0_v, rows1_v, acc_sh, sem0, sem1):
        cid = lax.axis_index("c")
        sid = lax.axis_index("s")
        wid = sid * _NC + cid
        _zero_rows(rows0_v, _CHUNK, K)
        _zero_acc_slice(rows0_v, acc_sh, sid)
        plsc.subcore_barrier()

        def stage(j, rows_v, sem):
            pltpu.make_async_copy(y_hbm.at[sidx_v.at[j]], rows_v, sem).wait()

            def scale(g, c2):
                w16 = w_v[j, pl.ds(g * 16, 16)]
                for l in range(16):
                    ws = jnp.full((16,), w16[l], dtype=jnp.float32)
                    i = g * 16 + l
                    for k in range(K):
                        rows_v[i, pl.ds(k * 16, 16)] = (
                            rows_v[i, pl.ds(k * 16, 16)] * ws)
                return c2

            lax.fori_loop(0, _CHUNK // 16, scale, 0)
            pltpu.sync_copy(rows_v, acc_sh.at[didx_v.at[j]], add=True)

            @pl.when(j + 2 < _SB)
            def _():
                pltpu.make_async_copy(
                    y_hbm.at[sidx_v.at[j + 2]], rows_v, sem).start()

        def block(b, cb):
            pltpu.sync_copy(src_hbm.at[wid, b], sidx_v)
            pltpu.sync_copy(dst_hbm.at[wid, b], didx_v)
            pltpu.sync_copy(w_hbm.at[wid, b], w_v)
            pltpu.make_async_copy(y_hbm.at[sidx_v.at[0]], rows0_v, sem0).start()
            pltpu.make_async_copy(y_hbm.at[sidx_v.at[1]], rows1_v, sem1).start()

            def pair(t, c):
                stage(2 * t, rows0_v, sem0)

                @pl.when(2 * t + 1 < _SB)
                def _():
                    stage(2 * t + 1, rows1_v, sem1)

                return c

            lax.fori_loop(0, (_SB + 1) // 2, pair, 0)
            return cb

        lax.fori_loop(0, _NSB, block, 0)
        plsc.subcore_barrier()
        rbase = sid * _RPT
        pltpu.sync_copy(acc_sh.at[pl.ds(rbase, _RPT)],
                        out_hbm.at[cid, pl.ds(rbase, _RPT)])

    return prop


_deg_call = _make_deg()
_prop128 = _make_prop(128)
_prop32 = _make_prop(32)


def _tc1(degacc, Z):
    def body(dega_ref, z_ref, dinv_ref, y1_ref):
        A = dega_ref[...]
        deg = (jnp.sum(A[0], axis=1) + jnp.sum(A[1], axis=1)) * (1.0 / 16.0) + 1.0
        dinv = lax.rsqrt(deg)[:, None]
        dinv_ref[...] = dinv
        y1_ref[...] = dinv * z_ref[...]

    return pl.pallas_call(
        body,
        grid=(_N // _RB,),
        in_specs=[pl.BlockSpec((2, _RB, 16), lambda i: (0, i, 0)),
                  pl.BlockSpec((_RB, 128), lambda i: (i, 0))],
        out_specs=[pl.BlockSpec((_RB, 1), lambda i: (i, 0)),
                   pl.BlockSpec((_RB, 128), lambda i: (i, 0))],
        out_shape=[jax.ShapeDtypeStruct((_N, 1), jnp.float32),
                   jax.ShapeDtypeStruct((_N, 128), jnp.float32)],
    )(degacc, Z)


def _tc2(p1, y1, dinv, W1, b1r, W2):
    def body(p_ref, y_ref, d_ref, w1_ref, b1_ref, w2_ref, y2_ref):
        dv = d_ref[...]
        pre = dv * (p_ref[0] + p_ref[1] + y_ref[...])
        H = jnp.maximum(
            jnp.dot(pre, w1_ref[...], preferred_element_type=jnp.float32)
            + b1_ref[...], 0.0)
        G = jnp.dot(H, w2_ref[...], preferred_element_type=jnp.float32)
        y2_ref[...] = dv * G

    return pl.pallas_call(
        body,
        grid=(_N // _RB,),
        in_specs=[pl.BlockSpec((2, _RB, 128), lambda i: (0, i, 0)),
                  pl.BlockSpec((_RB, 128), lambda i: (i, 0)),
                  pl.BlockSpec((_RB, 1), lambda i: (i, 0)),
                  pl.BlockSpec((128, 256), lambda i: (0, 0)),
                  pl.BlockSpec((1, 256), lambda i: (0, 0)),
                  pl.BlockSpec((256, 32), lambda i: (0, 0))],
        out_specs=pl.BlockSpec((_RB, 32), lambda i: (i, 0)),
        out_shape=jax.ShapeDtypeStruct((_N, 32), jnp.float32),
    )(p1, y1, dinv, W1, b1r, W2)


def _tc3(p2, y2, dinv, b2r):
    def body(p_ref, y_ref, d_ref, b2_ref, o_ref):
        B = d_ref[...] * (p_ref[0] + p_ref[1] + y_ref[...]) + b2_ref[...]
        m = jnp.max(B, axis=1, keepdims=True)
        e = jnp.exp(B - m)
        o_ref[...] = e / jnp.sum(e, axis=1, keepdims=True)

    return pl.pallas_call(
        body,
        grid=(_N // _RB,),
        in_specs=[pl.BlockSpec((2, _RB, 32), lambda i: (0, i, 0)),
                  pl.BlockSpec((_RB, 32), lambda i: (i, 0)),
                  pl.BlockSpec((_RB, 1), lambda i: (i, 0)),
                  pl.BlockSpec((1, 32), lambda i: (0, 0))],
        out_specs=pl.BlockSpec((_RB, 32), lambda i: (i, 0)),
        out_shape=jax.ShapeDtypeStruct((_N, 32), jnp.float32),
    )(p2, y2, dinv, b2r)


def kernel(Z, edge_index, edge_weight, W1, b1, W2, b2):
    src = edge_index[0].reshape(_NW, _NSB, _SB, _CHUNK)
    dst = edge_index[1].reshape(_NW, _NSB, _SB, _CHUNK)
    w = edge_weight.reshape(_NW, _NSB, _SB, _CHUNK)
    degacc = _deg_call(dst, w)
    dinv, y1 = _tc1(degacc, Z)
    p1 = _prop128(src, dst, w, y1)
    y2 = _tc2(p1, y1, dinv, W1, b1.reshape(1, -1), W2)
    p2 = _prop32(src, dst, w, y2)
    return _tc3(p2, y2, dinv, b2.reshape(1, -1))


# speedup vs baseline: 33.5201x; 1.1145x over previous
import functools

import jax
import jax.numpy as jnp
from jax import lax
from jax.experimental import pallas as pl
from jax.experimental.pallas import tpu as pltpu
from jax.experimental.pallas import tpu_sc as plsc

_N = 10000
_E = 320000
_NC = 2
_NS = 16
_NW = _NC * _NS
_EW = _E // _NW
_CHUNK = 80
_NCH = _EW // _CHUNK
_SB = 25
_NSB = _NCH // _SB
_NPAD = 10240
_RPT = _NPAD // _NS
_RB = 1000


def _zero_rows(rows_v, nrows, ncolv):
    z16 = jnp.zeros((16,), jnp.float32)

    def body(r, c):
        for k in range(ncolv):
            rows_v[r, pl.ds(k * 16, 16)] = z16
        return c

    lax.fori_loop(0, nrows, body, 0)


def _zero_acc_slice(rows_v, acc_sh, sid):
    rbase = sid * _RPT
    for q in range(_RPT // _CHUNK):
        pltpu.sync_copy(rows_v, acc_sh.at[pl.ds(rbase + q * _CHUNK, _CHUNK)])


def _make_deg():
    mesh = plsc.VectorSubcoreMesh(core_axis_name="c", subcore_axis_name="s")

    @functools.partial(
        pl.kernel,
        mesh=mesh,
        out_type=jax.ShapeDtypeStruct((_NC, _NPAD, 16), jnp.float32),
        scratch_types=[
            pltpu.VMEM((_SB, _CHUNK), jnp.int32),
            pltpu.VMEM((_SB, _CHUNK), jnp.float32),
            pltpu.VMEM((_CHUNK, 16), jnp.float32),
            pltpu.VMEM((_CHUNK, 16), jnp.float32),
            pltpu.VMEM_SHARED((_NPAD, 16), jnp.float32),
            pltpu.SemaphoreType.DMA,
            pltpu.SemaphoreType.DMA,
        ],
    )
    def deg_kernel(dst_hbm, w_hbm, out_hbm, didx_v, w_v,
                   rows0_v, rows1_v, acc_sh, sem0, sem1):
        cid = lax.axis_index("c")
        sid = lax.axis_index("s")
        wid = sid * _NC + cid
        _zero_rows(rows0_v, _CHUNK, 1)
        _zero_acc_slice(rows0_v, acc_sh, sid)
        plsc.subcore_barrier()

        rows = (rows0_v, rows1_v)
        sems = (sem0, sem1)

        def stage(j, b):
            def fill(g, c2):
                w16 = w_v[j, pl.ds(g * 16, 16)]
                for l in range(16):
                    rows[b][g * 16 + l, pl.ds(0, 16)] = jnp.full(
                        (16,), w16[l], dtype=jnp.float32)
                return c2

            @pl.when(j >= 2)
            def _():
                pltpu.make_async_copy(
                    rows[b], acc_sh.at[didx_v.at[j - 2]], sems[b]).wait()

            lax.fori_loop(0, _CHUNK // 16, fill, 0)
            pltpu.make_async_copy(
                rows[b], acc_sh.at[didx_v.at[j]], sems[b]).start(add=True)

        def block(b, cb):
            pltpu.sync_copy(dst_hbm.at[wid, b], didx_v)
            pltpu.sync_copy(w_hbm.at[wid, b], w_v)

            def pair(t, c):
                stage(2 * t, 0)

                @pl.when(2 * t + 1 < _SB)
                def _():
                    stage(2 * t + 1, 1)

                return c

            lax.fori_loop(0, (_SB + 1) // 2, pair, 0)
            pltpu.make_async_copy(
                rows[1], acc_sh.at[didx_v.at[_SB - 2]], sems[1]).wait()
            pltpu.make_async_copy(
                rows[0], acc_sh.at[didx_v.at[_SB - 1]], sems[0]).wait()
            return cb

        lax.fori_loop(0, _NSB, block, 0)
        plsc.subcore_barrier()
        rbase = sid * _RPT
        pltpu.sync_copy(acc_sh.at[pl.ds(rbase, _RPT)],
                        out_hbm.at[cid, pl.ds(rbase, _RPT)])

    return deg_kernel


def _make_prop(D):
    K = D // 16
    mesh = plsc.VectorSubcoreMesh(core_axis_name="c", subcore_axis_name="s")

    @functools.partial(
        pl.kernel,
        mesh=mesh,
        compiler_params=(None if D % 128 == 0 else
                         pltpu.CompilerParams(use_tc_tiling_on_sc=False)),
        out_type=jax.ShapeDtypeStruct((_NC, _NPAD, D), jnp.float32),
        scratch_types=[
            pltpu.VMEM((_SB, _CHUNK), jnp.int32),
            pltpu.VMEM((_SB, _CHUNK), jnp.int32),
            pltpu.VMEM((_SB, _CHUNK), jnp.float32),
            pltpu.VMEM((_CHUNK, D), jnp.float32),
            pltpu.VMEM((_CHUNK, D), jnp.float32),
            pltpu.VMEM((_CHUNK, D), jnp.float32),
            pltpu.VMEM_SHARED((_NPAD, D), jnp.float32),
            pltpu.SemaphoreType.DMA,
            pltpu.SemaphoreType.DMA,
            pltpu.SemaphoreType.DMA,
            pltpu.SemaphoreType.DMA,
            pltpu.SemaphoreType.DMA,
            pltpu.SemaphoreType.DMA,
        ],
    )
    def prop(src_hbm, dst_hbm, w_hbm, y_hbm, out_hbm,
             sidx_v, didx_v, w_v, rows0_v, rows1_v, rows2_v, acc_sh,
             g0, g1, g2, s0, s1, s2):
        cid = lax.axis_index("c")
        sid = lax.axis_index("s")
        wid = sid * _NC + cid
        _zero_rows(rows0_v, _CHUNK, K)
        _zero_acc_slice(rows0_v, acc_sh, sid)
        plsc.subcore_barrier()

        rows = (rows0_v, rows1_v, rows2_v)
        gsems = (g0, g1, g2)
        ssems = (s0, s1, s2)

        def gissue(j, b):
            pltpu.make_async_copy(
                y_hbm.at[sidx_v.at[j]], rows[b], gsems[b]).start()

        def stage(j, b, tail=False):
            pltpu.make_async_copy(
                y_hbm.at[sidx_v.at[j]], rows[b], gsems[b]).wait()

            def scale(g, c2):
                w16 = w_v[j, pl.ds(g * 16, 16)]
                for l in range(16):
                    ws = jnp.full((16,), w16[l], dtype=jnp.float32)
                    i = g * 16 + l
                    for k in range(K):
                        rows[b][i, pl.ds(k * 16, 16)] = (
                            rows[b][i, pl.ds(k * 16, 16)] * ws)
                return c2

            lax.fori_loop(0, _CHUNK // 16, scale, 0)
            pltpu.make_async_copy(
                rows[b], acc_sh.at[didx_v.at[j]], ssems[b]).start(add=True)
            if not tail:
                b2 = (b + 2) % 3

                @pl.when(j >= 1)
                def _():
                    pltpu.make_async_copy(
                        rows[b2], acc_sh.at[didx_v.at[j - 1]],
                        ssems[b2]).wait()

                @pl.when(j + 2 < _SB)
                def _():
                    gissue(j + 2, b2)

        def block(b, cb):
            pltpu.sync_copy(src_hbm.at[wid, b], sidx_v)
            pltpu.sync_copy(dst_hbm.at[wid, b], didx_v)
            pltpu.sync_copy(w_hbm.at[wid, b], w_v)
            gissue(0, 0)
            gissue(1, 1)

            def triple(t, c):
                stage(3 * t, 0)
                stage(3 * t + 1, 1)
                stage(3 * t + 2, 2)
                return c

            lax.fori_loop(0, _SB // 3, triple, 0)
            stage(_SB - 1, (_SB - 1) % 3, tail=True)
            pltpu.make_async_copy(
                rows[(_SB - 2) % 3], acc_sh.at[didx_v.at[_SB - 2]],
                ssems[(_SB - 2) % 3]).wait()
            pltpu.make_async_copy(
                rows[(_SB - 1) % 3], acc_sh.at[didx_v.at[_SB - 1]],
                ssems[(_SB - 1) % 3]).wait()
            return cb

        lax.fori_loop(0, _NSB, block, 0)
        plsc.subcore_barrier()
        rbase = sid * _RPT
        pltpu.sync_copy(acc_sh.at[pl.ds(rbase, _RPT)],
                        out_hbm.at[cid, pl.ds(rbase, _RPT)])

    return prop


_deg_call = _make_deg()
_prop128 = _make_prop(128)
_prop32 = _make_prop(32)


def _tc1(degacc, Z):
    def body(dega_ref, z_ref, dinv_ref, y1_ref):
        A = dega_ref[...]
        deg = (jnp.sum(A[0], axis=1) + jnp.sum(A[1], axis=1)) * (1.0 / 16.0) + 1.0
        dinv = lax.rsqrt(deg)[:, None]
        dinv_ref[...] = dinv
        y1_ref[...] = dinv * z_ref[...]

    return pl.pallas_call(
        body,
        grid=(_N // _RB,),
        in_specs=[pl.BlockSpec((2, _RB, 16), lambda i: (0, i, 0)),
                  pl.BlockSpec((_RB, 128), lambda i: (i, 0))],
        out_specs=[pl.BlockSpec((_RB, 1), lambda i: (i, 0)),
                   pl.BlockSpec((_RB, 128), lambda i: (i, 0))],
        out_shape=[jax.ShapeDtypeStruct((_N, 1), jnp.float32),
                   jax.ShapeDtypeStruct((_N, 128), jnp.float32)],
    )(degacc, Z)


def _tc2(p1, y1, dinv, W1, b1r, W2):
    def body(p_ref, y_ref, d_ref, w1_ref, b1_ref, w2_ref, y2_ref):
        dv = d_ref[...]
        pre = dv * (p_ref[0] + p_ref[1] + y_ref[...])
        H = jnp.maximum(
            jnp.dot(pre, w1_ref[...], preferred_element_type=jnp.float32)
            + b1_ref[...], 0.0)
        G = jnp.dot(H, w2_ref[...], preferred_element_type=jnp.float32)
        y2_ref[...] = dv * G

    return pl.pallas_call(
        body,
        grid=(_N // _RB,),
        in_specs=[pl.BlockSpec((2, _RB, 128), lambda i: (0, i, 0)),
                  pl.BlockSpec((_RB, 128), lambda i: (i, 0)),
                  pl.BlockSpec((_RB, 1), lambda i: (i, 0)),
                  pl.BlockSpec((128, 256), lambda i: (0, 0)),
                  pl.BlockSpec((1, 256), lambda i: (0, 0)),
                  pl.BlockSpec((256, 32), lambda i: (0, 0))],
        out_specs=pl.BlockSpec((_RB, 32), lambda i: (i, 0)),
        out_shape=jax.ShapeDtypeStruct((_N, 32), jnp.float32),
    )(p1, y1, dinv, W1, b1r, W2)


def _tc3(p2, y2, dinv, b2r):
    def body(p_ref, y_ref, d_ref, b2_ref, o_ref):
        B = d_ref[...] * (p_ref[0] + p_ref[1] + y_ref[...]) + b2_ref[...]
        m = jnp.max(B, axis=1, keepdims=True)
        e = jnp.exp(B - m)
        o_ref[...] = e / jnp.sum(e, axis=1, keepdims=True)

    return pl.pallas_call(
        body,
        grid=(_N // _RB,),
        in_specs=[pl.BlockSpec((2, _RB, 32), lambda i: (0, i, 0)),
                  pl.BlockSpec((_RB, 32), lambda i: (i, 0)),
                  pl.BlockSpec((_RB, 1), lambda i: (i, 0)),
                  pl.BlockSpec((1, 32), lambda i: (0, 0))],
        out_specs=pl.BlockSpec((_RB, 32), lambda i: (i, 0)),
        out_shape=jax.ShapeDtypeStruct((_N, 32), jnp.float32),
    )(p2, y2, dinv, b2r)


def kernel(Z, edge_index, edge_weight, W1, b1, W2, b2):
    src = edge_index[0].reshape(_NW, _NSB, _SB, _CHUNK)
    dst = edge_index[1].reshape(_NW, _NSB, _SB, _CHUNK)
    w = edge_weight.reshape(_NW, _NSB, _SB, _CHUNK)
    degacc = _deg_call(dst, w)
    dinv, y1 = _tc1(degacc, Z)
    p1 = _prop128(src, dst, w, y1)
    y2 = _tc2(p1, y1, dinv, W1, b1.reshape(1, -1), W2)
    p2 = _prop32(src, dst, w, y2)
    return _tc3(p2, y2, dinv, b2.reshape(1, -1))
